# baseline (speedup 1.0000x reference)
import jax
import jax.numpy as jnp
from jax import lax
from jax.experimental import pallas as pl
from jax.experimental.pallas import tpu as pltpu
from jax.experimental.pallas import tpu_sc as plsc

N = 16384
NUM_EVENT = 1000000
VP = 6.0
VS = 6.0 / 1.73
NC = 2
NS = 16
NW = NC * NS
S = N // NW
SCH = S // 128
L = 16


def _sc_body(eli_ref, eti_ref, sli_ref,
             eloc_ref, etime_ref, sloc_ref, s0_ref, s1_ref, td_ref,
             eli_v, eti_v, sli_v, loc_v, time_v, sg_v,
             s0_v, s1_v, td_v, sem):
    wid = lax.axis_index("s") * NC + lax.axis_index("c")

    pltpu.sync_copy(eli_ref.at[wid], eli_v)
    pltpu.sync_copy(eti_ref.at[wid], eti_v)
    pltpu.sync_copy(sli_ref.at[wid], sli_v)

    copies = []
    for j in range(6 * SCH):
        copies.append(pltpu.async_copy(
            eloc_ref.at[eli_v.at[j]], loc_v.at[pl.ds(j * 128, 128)], sem))
    for j in range(2 * SCH):
        copies.append(pltpu.async_copy(
            etime_ref.at[eti_v.at[j]], time_v.at[pl.ds(j * 128, 128)], sem))
    for j in range(3 * SCH):
        copies.append(pltpu.async_copy(
            sloc_ref.at[sli_v.at[j]], sg_v.at[pl.ds(j * 128, 128)], sem))
    for c in copies:
        c.wait()

    def step(it, carry):
        b = it * L
        sx = sg_v[pl.ds(b, L)]
        sy = sg_v[pl.ds(S + b, L)]
        sz = sg_v[pl.ds(2 * S + b, L)]
        x0 = loc_v[pl.ds(b, L)]
        y0 = loc_v[pl.ds(S + b, L)]
        z0 = loc_v[pl.ds(2 * S + b, L)]
        x1 = loc_v[pl.ds(3 * S + b, L)]
        y1 = loc_v[pl.ds(4 * S + b, L)]
        z1 = loc_v[pl.ds(5 * S + b, L)]
        dx0 = x0 - sx; dy0 = y0 - sy; dz0 = z0 - sz
        dx1 = x1 - sx; dy1 = y1 - sy; dz1 = z1 - sz
        s0_v[pl.ds(b, L)] = dx0 * dx0 + dy0 * dy0 + dz0 * dz0
        s1_v[pl.ds(b, L)] = dx1 * dx1 + dy1 * dy1 + dz1 * dz1
        td_v[pl.ds(b, L)] = time_v[pl.ds(b, L)] - time_v[pl.ds(S + b, L)]
        return carry

    lax.fori_loop(0, S // L, step, 0)
    pltpu.sync_copy(s0_v, s0_ref.at[pl.ds(wid * S, S)])
    pltpu.sync_copy(s1_v, s1_ref.at[pl.ds(wid * S, S)])
    pltpu.sync_copy(td_v, td_ref.at[pl.ds(wid * S, S)])


def _tc_body(s0_ref, s1_ref, td_ref, pt_ref, ptm_ref, pw_ref,
             pred_ref, loss_ref):
    d0 = jnp.sqrt(s0_ref[...])
    d1 = jnp.sqrt(s1_ref[...])
    inv_v = jnp.where(pt_ref[...] == 0,
                      jnp.float32(1.0 / VP), jnp.float32(1.0 / VS))
    pred = td_ref[...] + (d0 - d1) * inv_v
    pred_ref[...] = pred
    err = pred - ptm_ref[...]
    ae = jnp.abs(err)
    hub = jnp.where(ae < 1.0, 0.5 * err * err, ae - 0.5)
    loss_ref[0, 0] = jnp.sum(hub * pw_ref[...])


def _seg(x):
    return x.reshape(NW, SCH, 128)


@jax.jit
def kernel(station_index, event_index, phase_type, phase_time, phase_weight,
           event_loc_w, event_time_w, station_loc_w, station_dt_w):
    del station_dt_w
    ei = event_index.astype(jnp.int32)
    e0 = 3 * ei[:, 0]
    e1 = 3 * ei[:, 1]
    si3 = 3 * station_index.astype(jnp.int32)
    eli = jnp.concatenate(
        [_seg(e0), _seg(e0 + 1), _seg(e0 + 2),
         _seg(e1), _seg(e1 + 1), _seg(e1 + 2)], axis=1)
    eti = jnp.concatenate([_seg(ei[:, 0]), _seg(ei[:, 1])], axis=1)
    sli = jnp.concatenate([_seg(si3), _seg(si3 + 1), _seg(si3 + 2)], axis=1)
    eloc = event_loc_w.reshape(3 * NUM_EVENT)
    etime = event_time_w.reshape(NUM_EVENT)
    sloc = station_loc_w.reshape(300)

    mesh = plsc.VectorSubcoreMesh(core_axis_name="c", subcore_axis_name="s")
    sc = pl.kernel(
        _sc_body,
        out_type=(
            jax.ShapeDtypeStruct((N,), jnp.float32),
            jax.ShapeDtypeStruct((N,), jnp.float32),
            jax.ShapeDtypeStruct((N,), jnp.float32),
        ),
        mesh=mesh,
        scratch_types=[
            pltpu.VMEM((6 * SCH, 128), jnp.int32),
            pltpu.VMEM((2 * SCH, 128), jnp.int32),
            pltpu.VMEM((3 * SCH, 128), jnp.int32),
            pltpu.VMEM((6 * S,), jnp.float32),
            pltpu.VMEM((2 * S,), jnp.float32),
            pltpu.VMEM((3 * S,), jnp.float32),
            pltpu.VMEM((S,), jnp.float32),
            pltpu.VMEM((S,), jnp.float32),
            pltpu.VMEM((S,), jnp.float32),
            pltpu.SemaphoreType.DMA,
        ],
    )
    s0, s1, td = sc(eli, eti, sli, eloc, etime, sloc)

    pred2, loss = pl.pallas_call(
        _tc_body,
        out_shape=(
            jax.ShapeDtypeStruct((128, 128), jnp.float32),
            jax.ShapeDtypeStruct((1, 1), jnp.float32),
        ),
        out_specs=(
            pl.BlockSpec(memory_space=pltpu.VMEM),
            pl.BlockSpec(memory_space=pltpu.SMEM),
        ),
    )(s0.reshape(128, 128), s1.reshape(128, 128), td.reshape(128, 128),
      phase_type.astype(jnp.int32).reshape(128, 128),
      phase_time.reshape(128, 128), phase_weight.reshape(128, 128))
    return pred2.reshape(N), loss[0, 0]


# speedup vs baseline: 13.3548x; 13.3548x over previous
import jax
import jax.numpy as jnp
from jax import lax
from jax.experimental import pallas as pl
from jax.experimental.pallas import tpu as pltpu
from jax.experimental.pallas import tpu_sc as plsc

N = 16384
NUM_EVENT = 1000000
VP = 6.0
VS = 6.0 / 1.73
NC = 2
NS = 16
NW = NC * NS
S = N // NW
SCH = S // 128
L = 16


def _sc_body(eti_ref, sli_ref,
             ex_ref, ey_ref, ez_ref, et_ref, sx_ref, sy_ref, sz_ref,
             s0_ref, s1_ref, td_ref,
             eti_v, sli_v, loc_v, time_v, sg_v,
             s0_v, s1_v, td_v, sem):
    wid = lax.axis_index("s") * NC + lax.axis_index("c")

    pltpu.sync_copy(eti_ref.at[wid], eti_v)
    pltpu.sync_copy(sli_ref.at[wid], sli_v)

    CH = 256
    copies = []
    for seg, tbl in enumerate((ex_ref, ey_ref, ez_ref)):
        for j in range(2 * S // CH):
            copies.append(pltpu.async_copy(
                tbl.at[eti_v.at[pl.ds(j * CH, CH)]],
                loc_v.at[pl.ds(seg * 2 * S + j * CH, CH)], sem))
    for j in range(2 * S // CH):
        copies.append(pltpu.async_copy(
            et_ref.at[eti_v.at[pl.ds(j * CH, CH)]],
            time_v.at[pl.ds(j * CH, CH)], sem))
    for seg, tbl in enumerate((sx_ref, sy_ref, sz_ref)):
        for j in range(S // CH):
            copies.append(pltpu.async_copy(
                tbl.at[sli_v.at[pl.ds(j * CH, CH)]],
                sg_v.at[pl.ds(seg * S + j * CH, CH)], sem))
    for c in copies:
        c.wait()

    def step(it, carry):
        b = it * L
        sx = sg_v[pl.ds(b, L)]
        sy = sg_v[pl.ds(S + b, L)]
        sz = sg_v[pl.ds(2 * S + b, L)]
        x0 = loc_v[pl.ds(b, L)]
        x1 = loc_v[pl.ds(S + b, L)]
        y0 = loc_v[pl.ds(2 * S + b, L)]
        y1 = loc_v[pl.ds(3 * S + b, L)]
        z0 = loc_v[pl.ds(4 * S + b, L)]
        z1 = loc_v[pl.ds(5 * S + b, L)]
        dx0 = x0 - sx; dy0 = y0 - sy; dz0 = z0 - sz
        dx1 = x1 - sx; dy1 = y1 - sy; dz1 = z1 - sz
        s0_v[pl.ds(b, L)] = dx0 * dx0 + dy0 * dy0 + dz0 * dz0
        s1_v[pl.ds(b, L)] = dx1 * dx1 + dy1 * dy1 + dz1 * dz1
        td_v[pl.ds(b, L)] = time_v[pl.ds(b, L)] - time_v[pl.ds(S + b, L)]
        return carry

    lax.fori_loop(0, S // L, step, 0)
    pltpu.sync_copy(s0_v, s0_ref.at[pl.ds(wid * S, S)])
    pltpu.sync_copy(s1_v, s1_ref.at[pl.ds(wid * S, S)])
    pltpu.sync_copy(td_v, td_ref.at[pl.ds(wid * S, S)])


def _tc_body(s0_ref, s1_ref, td_ref, pt_ref, ptm_ref, pw_ref,
             pred_ref, loss_ref):
    d0 = jnp.sqrt(s0_ref[...])
    d1 = jnp.sqrt(s1_ref[...])
    inv_v = jnp.where(pt_ref[...] == 0,
                      jnp.float32(1.0 / VP), jnp.float32(1.0 / VS))
    pred = td_ref[...] + (d0 - d1) * inv_v
    pred_ref[...] = pred
    err = pred - ptm_ref[...]
    ae = jnp.abs(err)
    hub = jnp.where(ae < 1.0, 0.5 * err * err, ae - 0.5)
    loss_ref[0, 0] = jnp.sum(hub * pw_ref[...])


def _seg(x):
    return x.reshape(NW, SCH, 128)


@jax.jit
def kernel(station_index, event_index, phase_type, phase_time, phase_weight,
           event_loc_w, event_time_w, station_loc_w, station_dt_w):
    del station_dt_w
    ei = event_index.astype(jnp.int32)
    eti = jnp.concatenate([_seg(ei[:, 0]), _seg(ei[:, 1])],
                          axis=1).reshape(NW, 2 * S)
    sli = station_index.astype(jnp.int32).reshape(NW, S)
    elocT = event_loc_w.T
    ex, ey, ez = elocT[0], elocT[1], elocT[2]
    et = event_time_w.reshape(NUM_EVENT)
    slocT = station_loc_w.T
    sx, sy, sz = slocT[0], slocT[1], slocT[2]

    mesh = plsc.VectorSubcoreMesh(core_axis_name="c", subcore_axis_name="s")
    sc = pl.kernel(
        _sc_body,
        out_type=(
            jax.ShapeDtypeStruct((N,), jnp.float32),
            jax.ShapeDtypeStruct((N,), jnp.float32),
            jax.ShapeDtypeStruct((N,), jnp.float32),
        ),
        mesh=mesh,
        scratch_types=[
            pltpu.VMEM((2 * S,), jnp.int32),
            pltpu.VMEM((S,), jnp.int32),
            pltpu.VMEM((6 * S,), jnp.float32),
            pltpu.VMEM((2 * S,), jnp.float32),
            pltpu.VMEM((3 * S,), jnp.float32),
            pltpu.VMEM((S,), jnp.float32),
            pltpu.VMEM((S,), jnp.float32),
            pltpu.VMEM((S,), jnp.float32),
            pltpu.SemaphoreType.DMA,
        ],
    )
    s0, s1, td = sc(eti, sli, ex, ey, ez, et, sx, sy, sz)

    pred2, loss = pl.pallas_call(
        _tc_body,
        out_shape=(
            jax.ShapeDtypeStruct((128, 128), jnp.float32),
            jax.ShapeDtypeStruct((1, 1), jnp.float32),
        ),
        out_specs=(
            pl.BlockSpec(memory_space=pltpu.VMEM),
            pl.BlockSpec(memory_space=pltpu.SMEM),
        ),
    )(s0.reshape(128, 128), s1.reshape(128, 128), td.reshape(128, 128),
      phase_type.astype(jnp.int32).reshape(128, 128),
      phase_time.reshape(128, 128), phase_weight.reshape(128, 128))
    return pred2.reshape(N), loss[0, 0]


# speedup vs baseline: 27.9714x; 2.0945x over previous
import jax
import jax.numpy as jnp
from jax import lax
from jax.experimental import pallas as pl
from jax.experimental.pallas import tpu as pltpu
from jax.experimental.pallas import tpu_sc as plsc

N = 16384
NUM_EVENT = 1000000
VP = 6.0
VS = 6.0 / 1.73
NC = 2
NS = 16
NW = NC * NS
S = N // NW


def _sc_body(eti_ref, ex_ref, ey_ref, ez_ref, et_ref,
             x0_ref, x1_ref, y0_ref, y1_ref, z0_ref, z1_ref,
             t0_ref, t1_ref,
             eti_v, loc_v, time_v, sem):
    wid = lax.axis_index("s") * NC + lax.axis_index("c")

    pltpu.sync_copy(eti_ref.at[wid], eti_v)

    CH = 256
    copies = []
    for seg, tbl in enumerate((ex_ref, ey_ref, ez_ref)):
        for j in range(2 * S // CH):
            copies.append(pltpu.async_copy(
                tbl.at[eti_v.at[pl.ds(j * CH, CH)]],
                loc_v.at[pl.ds(seg * 2 * S + j * CH, CH)], sem))
    for j in range(2 * S // CH):
        copies.append(pltpu.async_copy(
            et_ref.at[eti_v.at[pl.ds(j * CH, CH)]],
            time_v.at[pl.ds(j * CH, CH)], sem))
    for c in copies:
        c.wait()

    outs = (x0_ref, x1_ref, y0_ref, y1_ref, z0_ref, z1_ref)
    for seg, ref in enumerate(outs):
        pltpu.sync_copy(loc_v.at[pl.ds(seg * S, S)],
                        ref.at[pl.ds(wid * S, S)])
    pltpu.sync_copy(time_v.at[pl.ds(0, S)], t0_ref.at[pl.ds(wid * S, S)])
    pltpu.sync_copy(time_v.at[pl.ds(S, S)], t1_ref.at[pl.ds(wid * S, S)])


def _tc_body(sid_ref, slx_ref, sly_ref, slz_ref,
             x0_ref, y0_ref, z0_ref, x1_ref, y1_ref, z1_ref,
             t0_ref, t1_ref, pt_ref, ptm_ref, pw_ref,
             pred_ref, loss_ref):
    sid = sid_ref[...]
    def lut(ref):
        t = jnp.broadcast_to(ref[...], (128, 128))
        return jnp.take_along_axis(t, sid, axis=1)
    sx = lut(slx_ref)
    sy = lut(sly_ref)
    sz = lut(slz_ref)
    dx0 = x0_ref[...] - sx
    dy0 = y0_ref[...] - sy
    dz0 = z0_ref[...] - sz
    dx1 = x1_ref[...] - sx
    dy1 = y1_ref[...] - sy
    dz1 = z1_ref[...] - sz
    s0 = dx0 * dx0 + dy0 * dy0 + dz0 * dz0
    s1 = dx1 * dx1 + dy1 * dy1 + dz1 * dz1
    inv_v = jnp.where(pt_ref[...] == 0,
                      jnp.float32(1.0 / VP), jnp.float32(1.0 / VS))
    pred = (t0_ref[...] - t1_ref[...]) + (jnp.sqrt(s0) - jnp.sqrt(s1)) * inv_v
    pred_ref[...] = pred
    err = pred - ptm_ref[...]
    ae = jnp.abs(err)
    hub = jnp.where(ae < 1.0, 0.5 * err * err, ae - 0.5)
    loss_ref[0, 0] = jnp.sum(hub * pw_ref[...])


def _seg(x):
    return x.reshape(NW, 4, 128)


@jax.jit
def kernel(station_index, event_index, phase_type, phase_time, phase_weight,
           event_loc_w, event_time_w, station_loc_w, station_dt_w):
    del station_dt_w
    ei = event_index.astype(jnp.int32)
    eti = jnp.concatenate([_seg(ei[:, 0]), _seg(ei[:, 1])],
                          axis=1).reshape(NW, 2 * S)
    elocT = event_loc_w.T
    ex, ey, ez = elocT[0], elocT[1], elocT[2]
    et = event_time_w[:, 0]

    mesh = plsc.VectorSubcoreMesh(core_axis_name="c", subcore_axis_name="s")
    sc = pl.kernel(
        _sc_body,
        out_type=tuple(
            jax.ShapeDtypeStruct((N,), jnp.float32) for _ in range(8)),
        mesh=mesh,
        scratch_types=[
            pltpu.VMEM((2 * S,), jnp.int32),
            pltpu.VMEM((6 * S,), jnp.float32),
            pltpu.VMEM((2 * S,), jnp.float32),
            pltpu.SemaphoreType.DMA,
        ],
    )
    x0, x1, y0, y1, z0, z1, t0, t1 = sc(eti, ex, ey, ez, et)

    slocT = station_loc_w.T
    spad = jnp.zeros((1, 128), jnp.float32)
    slx = spad.at[0, :100].set(slocT[0])
    sly = spad.at[0, :100].set(slocT[1])
    slz = spad.at[0, :100].set(slocT[2])

    r = lambda a: a.reshape(128, 128)
    pred2, loss = pl.pallas_call(
        _tc_body,
        out_shape=(
            jax.ShapeDtypeStruct((128, 128), jnp.float32),
            jax.ShapeDtypeStruct((1, 1), jnp.float32),
        ),
        out_specs=(
            pl.BlockSpec(memory_space=pltpu.VMEM),
            pl.BlockSpec(memory_space=pltpu.SMEM),
        ),
    )(station_index.astype(jnp.int32).reshape(128, 128), slx, sly, slz,
      r(x0), r(y0), r(z0), r(x1), r(y1), r(z1), r(t0), r(t1),
      phase_type.astype(jnp.int32).reshape(128, 128),
      phase_time.reshape(128, 128), phase_weight.reshape(128, 128))
    return pred2.reshape(N), loss[0, 0]


# speedup vs baseline: 32.6152x; 1.1660x over previous
import jax
import jax.numpy as jnp
from jax import lax
from jax.experimental import pallas as pl
from jax.experimental.pallas import tpu as pltpu
from jax.experimental.pallas import tpu_sc as plsc

N = 16384
NUM_EVENT = 1000000
VP = 6.0
VS = 6.0 / 1.73
NC = 2
NS = 16
NW = NC * NS
S = N // NW


def _sc_body(eti_ref, ex_ref, ey_ref, ez_ref, et_ref,
             x0_ref, x1_ref, y0_ref, y1_ref, z0_ref, z1_ref,
             t0_ref, t1_ref,
             eti_v, loc_v, time_v, sem):
    wid = lax.axis_index("s") * NC + lax.axis_index("c")

    pltpu.sync_copy(eti_ref.at[wid], eti_v)

    CH = 256
    copies = []
    for seg, tbl in enumerate((ex_ref, ey_ref, ez_ref)):
        for j in range(2 * S // CH):
            copies.append(pltpu.async_copy(
                tbl.at[eti_v.at[pl.ds(j * CH, CH)]],
                loc_v.at[pl.ds(seg * 2 * S + j * CH, CH)], sem))
    for j in range(2 * S // CH):
        copies.append(pltpu.async_copy(
            et_ref.at[eti_v.at[pl.ds(j * CH, CH)]],
            time_v.at[pl.ds(j * CH, CH)], sem))
    for c in copies:
        c.wait()

    outs = (x0_ref, x1_ref, y0_ref, y1_ref, z0_ref, z1_ref)
    for seg, ref in enumerate(outs):
        pltpu.sync_copy(loc_v.at[pl.ds(seg * S, S)],
                        ref.at[pl.ds(wid * S, S)])
    pltpu.sync_copy(time_v.at[pl.ds(0, S)], t0_ref.at[pl.ds(wid * S, S)])
    pltpu.sync_copy(time_v.at[pl.ds(S, S)], t1_ref.at[pl.ds(wid * S, S)])


def _tc_body(sid_ref, slx_ref, sly_ref, slz_ref,
             x0_ref, y0_ref, z0_ref, x1_ref, y1_ref, z1_ref,
             t0_ref, t1_ref, pt_ref, ptm_ref, pw_ref,
             pred_ref, loss_ref):
    sid = sid_ref[...]
    def lut(ref):
        t = jnp.broadcast_to(ref[...], (128, 128))
        return jnp.take_along_axis(t, sid, axis=1)
    sx = lut(slx_ref)
    sy = lut(sly_ref)
    sz = lut(slz_ref)
    dx0 = x0_ref[...] - sx
    dy0 = y0_ref[...] - sy
    dz0 = z0_ref[...] - sz
    dx1 = x1_ref[...] - sx
    dy1 = y1_ref[...] - sy
    dz1 = z1_ref[...] - sz
    s0 = dx0 * dx0 + dy0 * dy0 + dz0 * dz0
    s1 = dx1 * dx1 + dy1 * dy1 + dz1 * dz1
    inv_v = jnp.where(pt_ref[...] == 0,
                      jnp.float32(1.0 / VP), jnp.float32(1.0 / VS))
    pred = (t0_ref[...] - t1_ref[...]) + (jnp.sqrt(s0) - jnp.sqrt(s1)) * inv_v
    pred_ref[...] = pred
    err = pred - ptm_ref[...]
    ae = jnp.abs(err)
    hub = jnp.where(ae < 1.0, 0.5 * err * err, ae - 0.5)
    loss_ref[0, 0] = jnp.sum(hub * pw_ref[...])


def _seg(x):
    return x.reshape(NW, 4, 128)


@jax.jit
def kernel(station_index, event_index, phase_type, phase_time, phase_weight,
           event_loc_w, event_time_w, station_loc_w, station_dt_w):
    del station_dt_w
    ei = event_index.astype(jnp.int32)
    eti = jnp.concatenate([_seg(ei[:, 0]), _seg(ei[:, 1])],
                          axis=1).reshape(NW, 2 * S)
    catT = jnp.concatenate([event_loc_w, event_time_w], axis=1).T
    ex, ey, ez, et = catT[0], catT[1], catT[2], catT[3]

    mesh = plsc.VectorSubcoreMesh(core_axis_name="c", subcore_axis_name="s")
    sc = pl.kernel(
        _sc_body,
        out_type=tuple(
            jax.ShapeDtypeStruct((N,), jnp.float32) for _ in range(8)),
        mesh=mesh,
        scratch_types=[
            pltpu.VMEM((2 * S,), jnp.int32),
            pltpu.VMEM((6 * S,), jnp.float32),
            pltpu.VMEM((2 * S,), jnp.float32),
            pltpu.SemaphoreType.DMA,
        ],
    )
    x0, x1, y0, y1, z0, z1, t0, t1 = sc(eti, ex, ey, ez, et)

    slocT = station_loc_w.T
    spad = jnp.zeros((1, 128), jnp.float32)
    slx = spad.at[0, :100].set(slocT[0])
    sly = spad.at[0, :100].set(slocT[1])
    slz = spad.at[0, :100].set(slocT[2])

    r = lambda a: a.reshape(128, 128)
    pred2, loss = pl.pallas_call(
        _tc_body,
        out_shape=(
            jax.ShapeDtypeStruct((128, 128), jnp.float32),
            jax.ShapeDtypeStruct((1, 1), jnp.float32),
        ),
        out_specs=(
            pl.BlockSpec(memory_space=pltpu.VMEM),
            pl.BlockSpec(memory_space=pltpu.SMEM),
        ),
    )(station_index.astype(jnp.int32).reshape(128, 128), slx, sly, slz,
      r(x0), r(y0), r(z0), r(x1), r(y1), r(z1), r(t0), r(t1),
      phase_type.astype(jnp.int32).reshape(128, 128),
      phase_time.reshape(128, 128), phase_weight.reshape(128, 128))
    return pred2.reshape(N), loss[0, 0]
